# baseline (speedup 1.0000x reference)
import functools

import jax
import jax.numpy as jnp
from jax import lax
from jax.experimental import pallas as pl
from jax.experimental.pallas import tpu as pltpu
from jax.experimental.pallas import tpu_sc as plsc

N = 10000
E = 320000
F = 128
K = 64

NC = 2
NS = 16
NW = NC * NS
B = 80
EPW = E // NW
CHUNKS = EPW // B
CHT = E // B
NP = 10240
RPT = NP // NS
BE = 4000


def _dot_t(a, w):
    return lax.dot_general(a, w, (((1,), (1,)), ((), ())),
                           preferred_element_type=jnp.float32)


def _g_body(rbf_ref, wk_ref, out_ref):
    out_ref[...] = _dot_t(rbf_ref[...], wk_ref[...])


def _edge_matmul(rbf, Wk):
    return pl.pallas_call(
        _g_body,
        grid=(E // BE,),
        in_specs=[
            pl.BlockSpec((BE, K), lambda i: (i, 0)),
            pl.BlockSpec((F, K), lambda i: (0, 0)),
        ],
        out_specs=pl.BlockSpec((BE, F), lambda i: (i, 0)),
        out_shape=jax.ShapeDtypeStruct((E, F), jnp.float32),
    )(rbf, Wk)


def _node_body(x_ref, wi_ref, bi_ref, wj_ref, bj_ref, xi_ref, hj_ref):
    xv = x_ref[...]
    xi_ref[...] = _dot_t(xv, wi_ref[...]) + bi_ref[...]
    hj_ref[:N, :] = _dot_t(xv, wj_ref[...]) + bj_ref[...]
    hj_ref[N:, :] = jnp.zeros((NP - N, F), jnp.float32)


def _node_matmuls(x, Wi, bi, Wj, bj):
    return pl.pallas_call(
        _node_body,
        out_shape=(
            jax.ShapeDtypeStruct((N, F), jnp.float32),
            jax.ShapeDtypeStruct((NP, F), jnp.float32),
        ),
    )(x, Wi, bi.reshape(1, F), Wj, bj.reshape(1, F))


def _sc_body(g_hbm, hj_hbm, idxc_hbm, z_hbm, out_hbm,
             x0, x1, x2, x3, g0, g1, r0, r1, acc,
             isem0, isem1, isem2, isem3,
             lsem0, lsem1, gsem0, gsem1, ssem0, ssem1):
    c = lax.axis_index("c")
    s = lax.axis_index("s")
    wid = s * NC + c
    gbase = wid * EPW
    cbase = wid * CHUNKS
    nslice = pl.ds(s * RPT, RPT)
    xbufs = (x0, x1, x2, x3)
    gbufs = (g0, g1)
    rbufs = (r0, r1)
    isems = (isem0, isem1, isem2, isem3)
    lsems = (lsem0, lsem1)
    gsems = (gsem0, gsem1)
    ssems = (ssem0, ssem1)

    pltpu.sync_copy(z_hbm.at[nslice], acc.at[nslice])
    plsc.subcore_barrier()

    def start_idx(k, q):
        pltpu.async_copy(idxc_hbm.at[cbase + k], xbufs[q], isems[q])

    def wait_idx(q):
        pltpu.make_async_copy(idxc_hbm.at[0], xbufs[q], isems[q]).wait()

    def start_inputs(k, d, q):
        pltpu.async_copy(hj_hbm.at[xbufs[q].at[0]], rbufs[d], gsems[d])
        pltpu.async_copy(g_hbm.at[pl.ds(gbase + k * B, B)], gbufs[d], lsems[d])

    def wait_inputs(k, d, q):
        pltpu.make_async_copy(hj_hbm.at[xbufs[q].at[0]], rbufs[d],
                              gsems[d]).wait()
        pltpu.make_async_copy(g_hbm.at[pl.ds(gbase + k * B, B)], gbufs[d],
                              lsems[d]).wait()

    def start_scatter(d, q):
        pltpu.async_copy(rbufs[d], acc.at[xbufs[q].at[1]], ssems[d], add=True)

    def wait_scatter(d, q):
        pltpu.make_async_copy(rbufs[d], acc.at[xbufs[q].at[1]],
                              ssems[d]).wait()

    def multiply(d):
        @plsc.parallel_loop(0, B, 1, unroll=4)
        def _(i):
            for cc in range(F // 16):
                sli = pl.ds(cc * 16, 16)
                rbufs[d][i, sli] = rbufs[d][i, sli] * gbufs[d][i, sli]

    def do_chunk(k, q, d, first=False, start_next_idx=True, last=False):
        wait_inputs(k, d, q)
        if start_next_idx:
            start_idx(k + 2, (q + 2) % 4)
        if not first:
            wait_scatter(1 - d, (q + 3) % 4)
        if not last:
            wait_idx((q + 1) % 4)
            start_inputs(k + 1, 1 - d, (q + 1) % 4)
        multiply(d)
        start_scatter(d, q)

    start_idx(0, 0)
    start_idx(1, 1)
    wait_idx(0)
    start_inputs(0, 0, 0)
    do_chunk(0, 0, 0, first=True)

    def step(t, carry):
        kk = 1 + t * 4
        for b in range(4):
            do_chunk(kk + b, (1 + b) % 4, (1 + b) % 2)
        return carry

    lax.fori_loop(0, (CHUNKS - 5) // 4, step, 0)
    for k in range(CHUNKS - 4, CHUNKS):
        do_chunk(k, k % 4, k % 2,
                 start_next_idx=(k + 2 < CHUNKS), last=(k + 1 == CHUNKS))
    wait_scatter((CHUNKS - 1) % 2, (CHUNKS - 1) % 4)
    plsc.subcore_barrier()
    pltpu.sync_copy(acc.at[nslice], out_hbm.at[c, nslice])


def _sc_gather_scatter(g, hj, idxc, zeros_nf):
    mesh = plsc.VectorSubcoreMesh(core_axis_name="c", subcore_axis_name="s")
    f = pl.kernel(
        _sc_body,
        out_type=jax.ShapeDtypeStruct((NC, NP, F), jnp.float32),
        mesh=mesh,
        scratch_types=(
            [pltpu.VMEM((2, B), jnp.int32)] * 4
            + [pltpu.VMEM((B, F), jnp.float32)] * 4
            + [pltpu.VMEM_SHARED((NP, F), jnp.float32)]
            + [pltpu.SemaphoreType.DMA] * 10
        ),
    )
    return f(g, hj, idxc, zeros_nf)


def _fin_body(x_ref, xi_ref, p0_ref, w01, b01, w02, b02,
              w11, b11, w12, b12, wd, bd_, u_, out_ref):
    m = xi_ref[...] + p0_ref[0, :N, :] + p0_ref[1, :N, :]
    t = _dot_t(m, w01[...]) + b01[...]
    m = m + _dot_t(t, w02[...]) + b02[...]
    t = _dot_t(m, w11[...]) + b11[...]
    m = m + _dot_t(t, w12[...]) + b12[...]
    out_ref[...] = u_[...] * x_ref[...] + _dot_t(m, wd[...]) + bd_[...]


def _final(x, xi, p0, r0_W1, r0_b1, r0_W2, r0_b2,
           r1_W1, r1_b1, r1_W2, r1_b2, Wd, bd, u):
    return pl.pallas_call(
        _fin_body,
        out_shape=jax.ShapeDtypeStruct((N, F), jnp.float32),
    )(x, xi, p0, r0_W1, r0_b1.reshape(1, F), r0_W2, r0_b2.reshape(1, F),
      r1_W1, r1_b1.reshape(1, F), r1_W2, r1_b2.reshape(1, F),
      Wd, bd.reshape(1, F), u.reshape(1, F))


def kernel(x, rbf, idx_i, idx_j, Wk, Wi, bi, Wj, bj,
           r0_W1, r0_b1, r0_W2, r0_b2, r1_W1, r1_b1, r1_W2, r1_b2,
           Wd, bd, u):
    xi, hj = _node_matmuls(x, Wi, bi, Wj, bj)
    g0 = _edge_matmul(rbf, Wk)
    zeros_nf = jnp.zeros((NP, F), dtype=jnp.float32)
    idxc = jnp.stack([idx_j.reshape(CHT, B), idx_i.reshape(CHT, B)], axis=1)
    p0 = _sc_gather_scatter(g0, hj, idxc, zeros_nf)
    return _final(x, xi, p0, r0_W1, r0_b1, r0_W2, r0_b2,
                  r1_W1, r1_b1, r1_W2, r1_b2, Wd, bd, u)


# speedup vs baseline: 1.0205x; 1.0205x over previous
import jax
import jax.numpy as jnp
from jax import lax
from jax.experimental import pallas as pl
from jax.experimental.pallas import tpu as pltpu
from jax.experimental.pallas import tpu_sc as plsc

N = 10000
E = 320000
F = 128
K = 64

NC = 2
NS = 16
NW = NC * NS
EPW = E // NW
B = 80
CHUNKS = EPW // B
NP = 10240
RPT = NP // NS
BE = 4000
NB = 2000


def _dot_t(a, w):
    return lax.dot_general(a, w, (((1,), (1,)), ((), ())),
                           preferred_element_type=jnp.float32)


def _g_body(rbf_ref, wk_ref, x_ref, wi_ref, bi_ref, wj_ref, bj_ref,
            g_ref, xi_ref, hj_ref):
    g_ref[...] = _dot_t(rbf_ref[...], wk_ref[...])

    @pl.when(pl.program_id(0) == 0)
    def _():
        xv = x_ref[...]
        xi_ref[...] = _dot_t(xv, wi_ref[...]) + bi_ref[...]
        hj_ref[:N, :] = _dot_t(xv, wj_ref[...]) + bj_ref[...]
        hj_ref[N:, :] = jnp.zeros((NP - N, F), jnp.float32)


def _dense_in(rbf, Wk, x, Wi, bi, Wj, bj):
    cst = lambda i: (0, 0)
    return pl.pallas_call(
        _g_body,
        grid=(E // BE,),
        in_specs=[
            pl.BlockSpec((BE, K), lambda i: (i, 0)),
            pl.BlockSpec((F, K), cst),
            pl.BlockSpec((N, F), cst),
            pl.BlockSpec((F, F), cst),
            pl.BlockSpec((1, F), cst),
            pl.BlockSpec((F, F), cst),
            pl.BlockSpec((1, F), cst),
        ],
        out_specs=(
            pl.BlockSpec((BE, F), lambda i: (i, 0)),
            pl.BlockSpec((N, F), cst),
            pl.BlockSpec((NP, F), cst),
        ),
        out_shape=(
            jax.ShapeDtypeStruct((E, F), jnp.float32),
            jax.ShapeDtypeStruct((N, F), jnp.float32),
            jax.ShapeDtypeStruct((NP, F), jnp.float32),
        ),
    )(rbf, Wk, x, Wi, bi.reshape(1, F), Wj, bj.reshape(1, F))


def _sc_body(g_hbm, hj_hbm, idxi_hbm, idxj_hbm, z_hbm, out_hbm,
             ii0, ii1, ii2, ii3, ij0, ij1, ij2, ij3,
             g0, g1, r0, r1, acc,
             isem0, isem1, isem2, isem3,
             lsem0, lsem1, gsem0, gsem1, ssem0, ssem1):
    c = lax.axis_index("c")
    s = lax.axis_index("s")
    wid = s * NC + c
    ebase = wid * EPW
    nslice = pl.ds(s * RPT, RPT)
    iibufs = (ii0, ii1, ii2, ii3)
    ijbufs = (ij0, ij1, ij2, ij3)
    gbufs = (g0, g1)
    rbufs = (r0, r1)
    isems = (isem0, isem1, isem2, isem3)
    lsems = (lsem0, lsem1)
    gsems = (gsem0, gsem1)
    ssems = (ssem0, ssem1)

    pltpu.sync_copy(z_hbm.at[nslice], acc.at[nslice])
    plsc.subcore_barrier()

    def start_idx(k, q):
        off = pl.ds(ebase + k * B, B)
        pltpu.async_copy(idxj_hbm.at[off], ijbufs[q], isems[q])
        pltpu.async_copy(idxi_hbm.at[off], iibufs[q], isems[q])

    def wait_idx(q):
        pltpu.make_async_copy(idxj_hbm.at[pl.ds(0, B)], ijbufs[q],
                              isems[q]).wait()
        pltpu.make_async_copy(idxi_hbm.at[pl.ds(0, B)], iibufs[q],
                              isems[q]).wait()

    def start_inputs(k, d, q):
        pltpu.async_copy(hj_hbm.at[ijbufs[q]], rbufs[d], gsems[d])
        pltpu.async_copy(g_hbm.at[pl.ds(ebase + k * B, B)], gbufs[d], lsems[d])

    def wait_inputs(k, d, q):
        pltpu.make_async_copy(hj_hbm.at[ijbufs[q]], rbufs[d],
                              gsems[d]).wait()
        pltpu.make_async_copy(g_hbm.at[pl.ds(ebase + k * B, B)], gbufs[d],
                              lsems[d]).wait()

    def start_scatter(d, q):
        pltpu.async_copy(rbufs[d], acc.at[iibufs[q]], ssems[d], add=True)

    def wait_scatter(d, q):
        pltpu.make_async_copy(rbufs[d], acc.at[iibufs[q]],
                              ssems[d]).wait()

    start_idx(0, 0)
    start_idx(1, 1)
    wait_idx(0)
    start_inputs(0, 0, 0)

    def step(t, carry):
        kk = t * 4
        for b in range(4):
            k = kk + b
            d = b % 2

            @pl.when(k < CHUNKS)
            def _():
                wait_inputs(k, d, b)
                @pl.when(k + 2 < CHUNKS)
                def _():
                    start_idx(k + 2, (b + 2) % 4)

                @pl.when(k >= 1)
                def _():
                    wait_scatter(1 - d, (b + 3) % 4)

                @pl.when(k + 1 < CHUNKS)
                def _():
                    wait_idx((b + 1) % 4)
                    start_inputs(k + 1, 1 - d, (b + 1) % 4)

                @plsc.parallel_loop(0, B, 1, unroll=4)
                def _(i):
                    for cc in range(F // 16):
                        sli = pl.ds(cc * 16, 16)
                        rbufs[d][i, sli] = rbufs[d][i, sli] * gbufs[d][i, sli]

                start_scatter(d, b)

        return carry

    lax.fori_loop(0, (CHUNKS + 3) // 4, step, 0)
    wait_scatter((CHUNKS - 1) % 2, (CHUNKS - 1) % 4)
    plsc.subcore_barrier()
    pltpu.sync_copy(acc.at[nslice], out_hbm.at[c, nslice])


def _sc_gather_scatter(g, hj, idx_i, idx_j, zeros_nf):
    mesh = plsc.VectorSubcoreMesh(core_axis_name="c", subcore_axis_name="s")
    f = pl.kernel(
        _sc_body,
        out_type=jax.ShapeDtypeStruct((NC, NP, F), jnp.float32),
        mesh=mesh,
        scratch_types=(
            [pltpu.VMEM((B,), jnp.int32)] * 8
            + [pltpu.VMEM((B, F), jnp.float32)] * 4
            + [pltpu.VMEM_SHARED((NP, F), jnp.float32)]
            + [pltpu.SemaphoreType.DMA] * 10
        ),
    )
    return f(g, hj, idx_i, idx_j, zeros_nf)


def _fin_body(x_ref, xi_ref, p_ref, w01, b01, w02, b02,
              w11, b11, w12, b12, wd, bd_, u_, out_ref):
    m = xi_ref[...] + p_ref[0] + p_ref[1]
    t = _dot_t(m, w01[...]) + b01[...]
    m = m + _dot_t(t, w02[...]) + b02[...]
    t = _dot_t(m, w11[...]) + b11[...]
    m = m + _dot_t(t, w12[...]) + b12[...]
    out_ref[...] = u_[...] * x_ref[...] + _dot_t(m, wd[...]) + bd_[...]


def _final(x, xi, p0, r0_W1, r0_b1, r0_W2, r0_b2,
           r1_W1, r1_b1, r1_W2, r1_b2, Wd, bd, u):
    cst = lambda i: (0, 0)
    blk = lambda i: (i, 0)
    wspec = pl.BlockSpec((F, F), cst)
    bspec = pl.BlockSpec((1, F), cst)
    return pl.pallas_call(
        _fin_body,
        grid=(N // NB,),
        in_specs=[
            pl.BlockSpec((NB, F), blk),
            pl.BlockSpec((NB, F), blk),
            pl.BlockSpec((2, NB, F), lambda i: (0, i, 0)),
            wspec, bspec, wspec, bspec,
            wspec, bspec, wspec, bspec,
            wspec, bspec, bspec,
        ],
        out_specs=pl.BlockSpec((NB, F), blk),
        out_shape=jax.ShapeDtypeStruct((N, F), jnp.float32),
    )(x, xi, p0, r0_W1, r0_b1.reshape(1, F), r0_W2, r0_b2.reshape(1, F),
      r1_W1, r1_b1.reshape(1, F), r1_W2, r1_b2.reshape(1, F),
      Wd, bd.reshape(1, F), u.reshape(1, F))


def kernel(x, rbf, idx_i, idx_j, Wk, Wi, bi, Wj, bj,
           r0_W1, r0_b1, r0_W2, r0_b2, r1_W1, r1_b1, r1_W2, r1_b2,
           Wd, bd, u):
    g, xi, hj = _dense_in(rbf, Wk, x, Wi, bi, Wj, bj)
    zeros_nf = jnp.zeros((NP, F), dtype=jnp.float32)
    p0 = _sc_gather_scatter(g, hj, idx_i, idx_j, zeros_nf)
    return _final(x, xi, p0, r0_W1, r0_b1, r0_W2, r0_b2,
                  r1_W1, r1_b1, r1_W2, r1_b2, Wd, bd, u)


# speedup vs baseline: 1.0526x; 1.0314x over previous
import jax
import jax.numpy as jnp
from jax import lax
from jax.experimental import pallas as pl
from jax.experimental.pallas import tpu as pltpu
from jax.experimental.pallas import tpu_sc as plsc

N = 10000
E = 320000
F = 128
K = 64

NC = 2
NS = 16
NW = NC * NS
EPW = E // NW
B = 80
CHUNKS = EPW // B
NP = 10240
RPT = NP // NS
BE = 8000
NB = 2000


def _dot_t(a, w):
    return lax.dot_general(a, w, (((1,), (1,)), ((), ())),
                           preferred_element_type=jnp.float32)


def _g_body(rbf_ref, wk_ref, x_ref, wi_ref, bi_ref, wj_ref, bj_ref,
            g_ref, xi_ref, hj_ref):
    g_ref[...] = _dot_t(rbf_ref[...], wk_ref[...])

    @pl.when(pl.program_id(0) == 0)
    def _():
        xv = x_ref[...]
        xi_ref[...] = _dot_t(xv, wi_ref[...]) + bi_ref[...]
        hj_ref[:N, :] = _dot_t(xv, wj_ref[...]) + bj_ref[...]
        hj_ref[N:, :] = jnp.zeros((NP - N, F), jnp.float32)


def _dense_in(rbf, Wk, x, Wi, bi, Wj, bj):
    cst = lambda i: (0, 0)
    return pl.pallas_call(
        _g_body,
        grid=(E // BE,),
        in_specs=[
            pl.BlockSpec((BE, K), lambda i: (i, 0)),
            pl.BlockSpec((F, K), cst),
            pl.BlockSpec((N, F), cst),
            pl.BlockSpec((F, F), cst),
            pl.BlockSpec((1, F), cst),
            pl.BlockSpec((F, F), cst),
            pl.BlockSpec((1, F), cst),
        ],
        out_specs=(
            pl.BlockSpec((BE, F), lambda i: (i, 0)),
            pl.BlockSpec((N, F), cst),
            pl.BlockSpec((NP, F), cst),
        ),
        out_shape=(
            jax.ShapeDtypeStruct((E, F), jnp.float32),
            jax.ShapeDtypeStruct((N, F), jnp.float32),
            jax.ShapeDtypeStruct((NP, F), jnp.float32),
        ),
    )(rbf, Wk, x, Wi, bi.reshape(1, F), Wj, bj.reshape(1, F))


def _sc_body(g_hbm, hj_hbm, idxi_hbm, idxj_hbm, z_hbm, out_hbm,
             ii0, ii1, ii2, ii3, ij0, ij1, ij2, ij3,
             g0, g1, r0, r1, acc,
             isem0, isem1, isem2, isem3,
             lsem0, lsem1, gsem0, gsem1, ssem0, ssem1):
    c = lax.axis_index("c")
    s = lax.axis_index("s")
    wid = s * NC + c
    ebase = wid * EPW
    nslice = pl.ds(s * RPT, RPT)
    iibufs = (ii0, ii1, ii2, ii3)
    ijbufs = (ij0, ij1, ij2, ij3)
    gbufs = (g0, g1)
    rbufs = (r0, r1)
    isems = (isem0, isem1, isem2, isem3)
    lsems = (lsem0, lsem1)
    gsems = (gsem0, gsem1)
    ssems = (ssem0, ssem1)

    pltpu.sync_copy(z_hbm.at[nslice], acc.at[nslice])
    plsc.subcore_barrier()

    def start_idx(k, q):
        off = pl.ds(ebase + k * B, B)
        pltpu.async_copy(idxj_hbm.at[off], ijbufs[q], isems[q])
        pltpu.async_copy(idxi_hbm.at[off], iibufs[q], isems[q])

    def wait_idx(q):
        pltpu.make_async_copy(idxj_hbm.at[pl.ds(0, B)], ijbufs[q],
                              isems[q]).wait()
        pltpu.make_async_copy(idxi_hbm.at[pl.ds(0, B)], iibufs[q],
                              isems[q]).wait()

    def start_inputs(k, d, q):
        pltpu.async_copy(hj_hbm.at[ijbufs[q]], rbufs[d], gsems[d])
        pltpu.async_copy(g_hbm.at[pl.ds(ebase + k * B, B)], gbufs[d], lsems[d])

    def wait_inputs(k, d, q):
        pltpu.make_async_copy(hj_hbm.at[ijbufs[q]], rbufs[d],
                              gsems[d]).wait()
        pltpu.make_async_copy(g_hbm.at[pl.ds(ebase + k * B, B)], gbufs[d],
                              lsems[d]).wait()

    def start_scatter(d, q):
        pltpu.async_copy(rbufs[d], acc.at[iibufs[q]], ssems[d], add=True)

    def wait_scatter(d, q):
        pltpu.make_async_copy(rbufs[d], acc.at[iibufs[q]],
                              ssems[d]).wait()

    start_idx(0, 0)
    start_idx(1, 1)
    wait_idx(0)
    start_inputs(0, 0, 0)

    def step(t, carry):
        kk = t * 4
        for b in range(4):
            k = kk + b
            d = b % 2

            @pl.when(k < CHUNKS)
            def _():
                wait_inputs(k, d, b)
                @pl.when(k + 2 < CHUNKS)
                def _():
                    start_idx(k + 2, (b + 2) % 4)

                @pl.when(k >= 1)
                def _():
                    wait_scatter(1 - d, (b + 3) % 4)

                @pl.when(k + 1 < CHUNKS)
                def _():
                    wait_idx((b + 1) % 4)
                    start_inputs(k + 1, 1 - d, (b + 1) % 4)

                @plsc.parallel_loop(0, B, 1, unroll=4)
                def _(i):
                    for cc in range(F // 16):
                        sli = pl.ds(cc * 16, 16)
                        rbufs[d][i, sli] = rbufs[d][i, sli] * gbufs[d][i, sli]

                start_scatter(d, b)

        return carry

    lax.fori_loop(0, (CHUNKS + 3) // 4, step, 0)
    wait_scatter((CHUNKS - 1) % 2, (CHUNKS - 1) % 4)
    plsc.subcore_barrier()
    pltpu.sync_copy(acc.at[nslice], out_hbm.at[c, nslice])


def _sc_gather_scatter(g, hj, idx_i, idx_j, zeros_nf):
    mesh = plsc.VectorSubcoreMesh(core_axis_name="c", subcore_axis_name="s")
    f = pl.kernel(
        _sc_body,
        out_type=jax.ShapeDtypeStruct((NC, NP, F), jnp.float32),
        mesh=mesh,
        scratch_types=(
            [pltpu.VMEM((B,), jnp.int32)] * 8
            + [pltpu.VMEM((B, F), jnp.float32)] * 4
            + [pltpu.VMEM_SHARED((NP, F), jnp.float32)]
            + [pltpu.SemaphoreType.DMA] * 10
        ),
    )
    return f(g, hj, idx_i, idx_j, zeros_nf)


def _fin_body(x_ref, xi_ref, p_ref, w01, b01, w02, b02,
              w11, b11, w12, b12, wd, bd_, u_, out_ref):
    m = xi_ref[...] + p_ref[0] + p_ref[1]
    t = _dot_t(m, w01[...]) + b01[...]
    m = m + _dot_t(t, w02[...]) + b02[...]
    t = _dot_t(m, w11[...]) + b11[...]
    m = m + _dot_t(t, w12[...]) + b12[...]
    out_ref[...] = u_[...] * x_ref[...] + _dot_t(m, wd[...]) + bd_[...]


def _final(x, xi, p0, r0_W1, r0_b1, r0_W2, r0_b2,
           r1_W1, r1_b1, r1_W2, r1_b2, Wd, bd, u):
    cst = lambda i: (0, 0)
    blk = lambda i: (i, 0)
    wspec = pl.BlockSpec((F, F), cst)
    bspec = pl.BlockSpec((1, F), cst)
    return pl.pallas_call(
        _fin_body,
        grid=(N // NB,),
        in_specs=[
            pl.BlockSpec((NB, F), blk),
            pl.BlockSpec((NB, F), blk),
            pl.BlockSpec((2, NB, F), lambda i: (0, i, 0)),
            wspec, bspec, wspec, bspec,
            wspec, bspec, wspec, bspec,
            wspec, bspec, bspec,
        ],
        out_specs=pl.BlockSpec((NB, F), blk),
        out_shape=jax.ShapeDtypeStruct((N, F), jnp.float32),
    )(x, xi, p0, r0_W1, r0_b1.reshape(1, F), r0_W2, r0_b2.reshape(1, F),
      r1_W1, r1_b1.reshape(1, F), r1_W2, r1_b2.reshape(1, F),
      Wd, bd.reshape(1, F), u.reshape(1, F))


def kernel(x, rbf, idx_i, idx_j, Wk, Wi, bi, Wj, bj,
           r0_W1, r0_b1, r0_W2, r0_b2, r1_W1, r1_b1, r1_W2, r1_b2,
           Wd, bd, u):
    g, xi, hj = _dense_in(rbf, Wk, x, Wi, bi, Wj, bj)
    zeros_nf = jnp.zeros((NP, F), dtype=jnp.float32)
    p0 = _sc_gather_scatter(g, hj, idx_i, idx_j, zeros_nf)
    return _final(x, xi, p0, r0_W1, r0_b1, r0_W2, r0_b2,
                  r1_W1, r1_b1, r1_W2, r1_b2, Wd, bd, u)
